# baseline (speedup 1.0000x reference)
import jax
import jax.numpy as jnp
from jax.experimental import pallas as pl
from jax.experimental.pallas import tpu as pltpu

BATCH = 2
HIDDEN = 32


def _probe_kernel(ids_ref, wemb_ref, ip_ref, fp_ref, bp_ref,
                  logits_ref, pooled_ref):
    pooled_ref[...] = fp_ref[0:BATCH, 0:HIDDEN] * 0.0
    logits_ref[...] = jnp.zeros((BATCH, 1), jnp.float32) + ids_ref[0].astype(jnp.float32) * 0.0
    del ip_ref, bp_ref, wemb_ref


def kernel(word_emb, pos_emb, type_emb, emb_ln_g, emb_ln_b, qkv_w, qkv_b,
           o_w, o_b, attn_ln_g, attn_ln_b, ffn_w1, ffn_b1, ffn_w2, ffn_b2,
           out_ln_g, out_ln_b, pool_w, pool_b, cls_w, cls_b,
           input_ids, attention_mask, token_type_ids):
    ids = input_ids.reshape(-1)
    wemb_t = word_emb.T
    intpack = jnp.concatenate([token_type_ids, attention_mask], axis=0)

    def p96(x):
        x = x.reshape(-1, x.shape[-1])
        return jnp.pad(x, ((0, 0), (0, 96 - x.shape[-1])))

    f32pack = jnp.concatenate([
        p96(pos_emb), p96(type_emb), p96(emb_ln_g), p96(emb_ln_b),
        p96(qkv_b), p96(o_b), p96(attn_ln_g), p96(attn_ln_b),
        p96(ffn_b1), p96(ffn_b2), p96(out_ln_g), p96(out_ln_b),
        p96(pool_w), p96(pool_b), p96(cls_w), p96(cls_b)], axis=0)
    bf16pack = jnp.concatenate([
        p96(qkv_w), p96(o_w), p96(ffn_w1), p96(ffn_w2)], axis=0)

    def vmem(shape):
        return pl.BlockSpec(shape, lambda *_: (0,) * len(shape))

    grid_spec = pltpu.PrefetchScalarGridSpec(
        num_scalar_prefetch=1,
        grid=(1,),
        in_specs=[
            pl.BlockSpec(memory_space=pltpu.MemorySpace.HBM),
            vmem(intpack.shape), vmem(f32pack.shape), vmem(bf16pack.shape),
        ],
        out_specs=(vmem((BATCH, 1)), vmem((BATCH, HIDDEN))),
        scratch_shapes=[],
    )

    logits, pooled = pl.pallas_call(
        _probe_kernel,
        grid_spec=grid_spec,
        out_shape=(jax.ShapeDtypeStruct((BATCH, 1), jnp.float32),
                   jax.ShapeDtypeStruct((BATCH, HIDDEN), jnp.float32)),
        compiler_params=pltpu.CompilerParams(
            dimension_semantics=("arbitrary",),
            disable_bounds_checks=True),
    )(ids, wemb_t, intpack, f32pack, bf16pack)
    return logits, pooled


# speedup vs baseline: 2.2018x; 2.2018x over previous
import jax
import jax.numpy as jnp
from jax.experimental import pallas as pl
from jax.experimental.pallas import tpu as pltpu

BATCH = 2
SEQ = 8
HIDDEN = 32
NUM_HEADS = 2
HEAD_DIM = HIDDEN // NUM_HEADS
INTERMEDIATE = 64
NUM_LAYERS = 2
LN_EPS = 1e-12
M = BATCH * SEQ
A = BATCH * NUM_HEADS * SEQ


def _layernorm(x, g, b):
    mu = jnp.mean(x, axis=-1, keepdims=True)
    var = jnp.mean((x - mu) ** 2, axis=-1, keepdims=True)
    return (x - mu) * jax.lax.rsqrt(var + LN_EPS) * g + b


def _fused_kernel(ids_ref,
                  wemb_ref,
                  *rest):
    hbm = rest[:22]
    logits_ref, pooled_ref = rest[22], rest[23]
    emb3 = rest[24]
    vb = rest[25:47]
    gsem, wsem = rest[47], rest[48]

    (tt_ref, am_ref, pos_ref, type_ref, eg_ref, eb_ref,
     qkvw_ref, qkvb_ref, ow_ref, ob_ref, ag_ref, ab_ref,
     w1_ref, b1_ref, w2_ref, b2_ref, og_ref, ogb_ref,
     pw_ref, pb_ref, cw_ref, cb_ref) = vb

    copies = [
        pltpu.make_async_copy(
            wemb_ref.at[:, pl.ds(pl.multiple_of((ids_ref[t] >> 7) << 7, 128),
                                 128)],
            emb3.at[:, pl.ds(128 * t, 128)],
            gsem)
        for t in range(M)
    ]
    for c in copies:
        c.start()
    wcopies = [pltpu.make_async_copy(h, v, wsem) for h, v in zip(hbm, vb)]
    for c in wcopies:
        c.start()
    for c in wcopies:
        c.wait()

    p8 = pos_ref[0:SEQ, :]
    posm = jnp.concatenate([p8] * BATCH, axis=0)
    te0 = type_ref[0:1, :]
    delta = type_ref[1:2, :] - te0
    ttf = tt_ref[...].astype(jnp.float32)
    ttrep = jnp.concatenate(
        [jnp.broadcast_to(ttf[b:b + 1, :], (SEQ, SEQ)) for b in range(BATCH)],
        axis=0)
    lane = jax.lax.broadcasted_iota(jnp.int32, (M, SEQ), 1)
    srow = jax.lax.broadcasted_iota(jnp.int32, (M, SEQ), 0) & (SEQ - 1)
    ttcol = jnp.sum(jnp.where(lane == srow, ttrep, 0.0), axis=1,
                    keepdims=True)
    typem = te0 + ttcol * delta

    amf = (1.0 - am_ref[...].astype(jnp.float32)) * -10000.0
    m_all = jnp.concatenate([amf[0:1, :], amf[0:1, :],
                             amf[1:2, :], amf[1:2, :]], axis=1)
    r8 = jax.lax.broadcasted_iota(jnp.int32, (A, A), 0) >> 3
    c8 = jax.lax.broadcasted_iota(jnp.int32, (A, A), 1) >> 3
    blockm = jnp.where(r8 == c8, 0.0, -30000.0)

    for c in copies:
        c.wait()
    word_cols = [
        pltpu.roll(emb3[:, 128 * t:128 * t + 128], -(ids_ref[t] & 127),
                   axis=1)[:, 0:1]
        for t in range(M)
    ]
    wordm_t = jnp.concatenate(word_cols, axis=1)
    wordm = wordm_t.T

    x = _layernorm(wordm + posm + typem, eg_ref[...], eb_ref[...])

    scale = 1.0 / (HEAD_DIM ** 0.5)
    D = HEAD_DIM
    for l in range(NUM_LAYERS):
        qkv = (jnp.dot(x.astype(jnp.bfloat16), qkvw_ref[l],
                       preferred_element_type=jnp.float32) + qkvb_ref[l])
        def pack(base):
            return jnp.concatenate(
                [qkv[b * SEQ:(b + 1) * SEQ, base + h * D:base + (h + 1) * D]
                 for b in range(BATCH) for h in range(NUM_HEADS)], axis=0)
        q_all = pack(0)
        k_all = pack(HIDDEN)
        v_all = pack(2 * HIDDEN)
        s = jax.lax.dot_general(
            q_all.astype(jnp.bfloat16), k_all.astype(jnp.bfloat16),
            (((1,), (1,)), ((), ())),
            preferred_element_type=jnp.float32) * scale + m_all + blockm
        s = s - jnp.max(s, axis=-1, keepdims=True)
        p = jnp.exp(s)
        p = p * pl.reciprocal(jnp.sum(p, axis=-1, keepdims=True), approx=True)
        ctx_all = jnp.dot(p.astype(jnp.bfloat16), v_all.astype(jnp.bfloat16),
                          preferred_element_type=jnp.float32)
        ctx = jnp.concatenate(
            [jnp.concatenate(
                [ctx_all[(b * NUM_HEADS + h) * SEQ:(b * NUM_HEADS + h + 1) * SEQ, :]
                 for h in range(NUM_HEADS)], axis=1)
             for b in range(BATCH)], axis=0)

        attn = (jnp.dot(ctx.astype(jnp.bfloat16), ow_ref[l],
                        preferred_element_type=jnp.float32) + ob_ref[l])
        x = _layernorm(x + attn, ag_ref[l], ab_ref[l])

        h1 = (jnp.dot(x.astype(jnp.bfloat16), w1_ref[l],
                      preferred_element_type=jnp.float32) + b1_ref[l])
        h1 = jax.nn.gelu(h1, approximate=True)
        ffn = (jnp.dot(h1.astype(jnp.bfloat16), w2_ref[l],
                       preferred_element_type=jnp.float32) + b2_ref[l])
        x = _layernorm(x + ffn, og_ref[l], ogb_ref[l])

    cls_tok = jnp.concatenate([x[b * SEQ:b * SEQ + 1, :] for b in range(BATCH)],
                              axis=0)
    pooled = jnp.tanh(jnp.dot(cls_tok, pw_ref[...],
                              preferred_element_type=jnp.float32) + pb_ref[...])
    pooled_ref[...] = pooled
    logits_ref[...] = (jnp.dot(pooled, cw_ref[...],
                               preferred_element_type=jnp.float32) + cb_ref[...])


def kernel(word_emb, pos_emb, type_emb, emb_ln_g, emb_ln_b, qkv_w, qkv_b,
           o_w, o_b, attn_ln_g, attn_ln_b, ffn_w1, ffn_b1, ffn_w2, ffn_b2,
           out_ln_g, out_ln_b, pool_w, pool_b, cls_w, cls_b,
           input_ids, attention_mask, token_type_ids):
    ids = input_ids.reshape(-1)
    wemb_t = word_emb.T

    operands = (token_type_ids, attention_mask, pos_emb, type_emb,
                emb_ln_g, emb_ln_b, qkv_w, qkv_b, o_w, o_b,
                attn_ln_g, attn_ln_b, ffn_w1, ffn_b1, ffn_w2, ffn_b2,
                out_ln_g, out_ln_b, pool_w, pool_b, cls_w, cls_b)

    def vmem(shape):
        return pl.BlockSpec(shape, lambda *_: (0,) * len(shape))

    grid_spec = pltpu.PrefetchScalarGridSpec(
        num_scalar_prefetch=1,
        grid=(1,),
        in_specs=[pl.BlockSpec(memory_space=pltpu.MemorySpace.HBM)] * 23,
        out_specs=(vmem((BATCH, 1)), vmem((BATCH, HIDDEN))),
        scratch_shapes=[
            pltpu.VMEM((HIDDEN, 128 * M), jnp.float32),
        ] + [pltpu.VMEM(o.shape, o.dtype) for o in operands] + [
            pltpu.SemaphoreType.DMA,
            pltpu.SemaphoreType.DMA,
        ],
    )

    logits, pooled = pl.pallas_call(
        _fused_kernel,
        grid_spec=grid_spec,
        out_shape=(jax.ShapeDtypeStruct((BATCH, 1), jnp.float32),
                   jax.ShapeDtypeStruct((BATCH, HIDDEN), jnp.float32)),
        compiler_params=pltpu.CompilerParams(
            dimension_semantics=("arbitrary",),
            disable_bounds_checks=True),
    )(ids, wemb_t, *operands)
    return logits, pooled
